# Initial kernel scaffold; baseline (speedup 1.0000x reference)
import jax
import jax.numpy as jnp
from jax.experimental import pallas as pl


def kernel(cpt_idx, cpt_vec):
    raise NotImplementedError("write your pallas kernel here")


# speedup vs baseline: 1.4240x; 1.4240x over previous
import functools
import jax
import jax.numpy as jnp
from jax import lax
from jax.experimental import pallas as pl
from jax.experimental.pallas import tpu as pltpu
from jax.experimental.pallas import tpu_sc as plsc

_BLK = 128


@functools.lru_cache(maxsize=None)
def _make_gather(num_idx, vocab, emb):
    info = plsc.get_sparse_core_info()
    nc, ns = info.num_cores, info.num_subcores
    nw = nc * ns
    assert num_idx % (nw * _BLK) == 0
    nblk = num_idx // (nw * _BLK)
    mesh = plsc.VectorSubcoreMesh(core_axis_name="c", subcore_axis_name="s")

    @functools.partial(
        pl.kernel,
        mesh=mesh,
        out_type=jax.ShapeDtypeStruct((nw, nblk, _BLK, emb), jnp.float32),
        scratch_types=[
            pltpu.VMEM((nblk, _BLK), jnp.int32),
            pltpu.VMEM((2, _BLK, emb), jnp.float32),
            pltpu.SemaphoreType.DMA,
            pltpu.SemaphoreType.DMA,
        ],
        compiler_params=pltpu.CompilerParams(use_tc_tiling_on_sc=False),
    )
    def gather_kernel(idx_hbm, table_hbm, out_hbm, idx_v, rows_v, gsem, osem):
        wid = lax.axis_index("s") * nc + lax.axis_index("c")
        pltpu.sync_copy(idx_hbm.at[wid], idx_v)
        pltpu.async_copy(table_hbm.at[idx_v.at[0]], rows_v.at[0], gsem)

        def body(j, _):
            slot = lax.rem(j, 2)
            nxt = lax.rem(j + 1, 2)

            @pl.when(j + 1 < nblk)
            def _start_next():
                @pl.when(j >= 1)
                def _drain_prev_out():
                    pltpu.make_async_copy(
                        rows_v.at[nxt], out_hbm.at[wid, j - 1], osem
                    ).wait()

                pltpu.async_copy(
                    table_hbm.at[idx_v.at[j + 1]], rows_v.at[nxt], gsem
                )

            pltpu.make_async_copy(
                table_hbm.at[idx_v.at[j]], rows_v.at[slot], gsem
            ).wait()
            pltpu.async_copy(rows_v.at[slot], out_hbm.at[wid, j], osem)
            return 0

        lax.fori_loop(0, nblk, body, 0)
        last = nblk - 1
        pltpu.make_async_copy(
            rows_v.at[lax.rem(last, 2)], out_hbm.at[wid, last], osem
        ).wait()

        @pl.when(nblk >= 2)
        def _():
            pltpu.make_async_copy(
                rows_v.at[lax.rem(last + 1, 2)], out_hbm.at[wid, last - 1], osem
            ).wait()

    return gather_kernel


def kernel(cpt_idx, cpt_vec):
    b, cpt_num = cpt_idx.shape
    vocab, emb = cpt_vec.shape
    num_idx = b * cpt_num
    fn = _make_gather(num_idx, vocab, emb)
    nw_nblk = num_idx // _BLK
    idx3 = cpt_idx.astype(jnp.int32).reshape(32, nw_nblk // 32, _BLK)
    out = fn(idx3, cpt_vec)
    return out.reshape(b, cpt_num, emb)
